# Initial kernel scaffold; baseline (speedup 1.0000x reference)
import jax
import jax.numpy as jnp
from jax.experimental import pallas as pl


def kernel(x, weight):
    raise NotImplementedError("write your pallas kernel here")


# speedup vs baseline: 1.0684x; 1.0684x over previous
import functools

import jax
import jax.numpy as jnp
from jax.experimental import pallas as pl
from jax.experimental.pallas import tpu as pltpu

_K = 8192
_D = 256
_ROWS = 256


def _vq_tile(f_ref, w_ref, enc_ref, idx_ref, q_ref, ls_ref):
    f = f_ref[...]
    w = w_ref[...]
    x2 = jnp.sum(f * f, axis=1, keepdims=True)
    w2 = jnp.sum(w * w, axis=1)
    mm = jax.lax.dot_general(
        f, w, dimension_numbers=(((1,), (1,)), ((), ())),
        preferred_element_type=jnp.float32)
    d = (x2 + w2[None, :]) - 2.0 * mm
    dmin = jnp.min(d, axis=1, keepdims=True)
    kiota = jax.lax.broadcasted_iota(jnp.int32, d.shape, 1)
    idx = jnp.min(jnp.where(d == dmin, kiota, _K), axis=1)
    enc = (kiota == idx[:, None]).astype(jnp.float32)
    enc_ref[...] = enc
    idx_ref[...] = idx.reshape(1, 1, _ROWS)
    q = jax.lax.dot_general(
        enc.astype(jnp.bfloat16), w.astype(jnp.bfloat16),
        dimension_numbers=(((1,), (0,)), ((), ())),
        preferred_element_type=jnp.float32)
    q_ref[...] = q
    part = jnp.sum((q - f) ** 2, keepdims=True).reshape(1, 1)
    ls_ref[...] = jnp.broadcast_to(part, (1, 1, 128))


@functools.partial(jax.jit, static_argnames=())
def kernel(x, weight):
    n = x.shape[0] * x.shape[2] * x.shape[3]
    grid = n // _ROWS
    xp = jnp.transpose(x, (0, 2, 3, 1))
    flat = xp.reshape(n, _D)
    enc, idx3, q, ls = pl.pallas_call(
        _vq_tile,
        grid=(grid,),
        in_specs=[
            pl.BlockSpec((_ROWS, _D), lambda i: (i, 0)),
            pl.BlockSpec((_K, _D), lambda i: (0, 0)),
        ],
        out_specs=[
            pl.BlockSpec((_ROWS, _K), lambda i: (i, 0)),
            pl.BlockSpec((1, 1, _ROWS), lambda i: (i, 0, 0)),
            pl.BlockSpec((_ROWS, _D), lambda i: (i, 0)),
            pl.BlockSpec((1, 1, 128), lambda i: (i, 0, 0)),
        ],
        out_shape=[
            jax.ShapeDtypeStruct((n, _K), jnp.float32),
            jax.ShapeDtypeStruct((grid, 1, _ROWS), jnp.int32),
            jax.ShapeDtypeStruct((n, _D), jnp.float32),
            jax.ShapeDtypeStruct((grid, 1, 128), jnp.float32),
        ],
        compiler_params=pltpu.CompilerParams(
            dimension_semantics=("parallel",),
        ),
    )(flat, weight)
    loss = jnp.sum(ls[:, 0, 0]) * (2.0 / (n * _D))
    quantized_st = jnp.transpose(q.reshape(xp.shape), (0, 3, 1, 2))
    return loss, quantized_st, enc, idx3.reshape(n)


# speedup vs baseline: 1.0875x; 1.0179x over previous
import functools

import jax
import jax.numpy as jnp
from jax.experimental import pallas as pl
from jax.experimental.pallas import tpu as pltpu

_K = 8192
_D = 256
_ROWS = 256


def _vq_tile(f_ref, w_ref, enc_ref, idx_ref, q_ref, ls_ref):
    f = f_ref[...]
    w = w_ref[...]
    x2 = jnp.sum(f * f, axis=1, keepdims=True)
    w2 = jnp.sum(w * w, axis=1)
    mm = jax.lax.dot_general(
        f.astype(jnp.bfloat16), w.astype(jnp.bfloat16),
        dimension_numbers=(((1,), (1,)), ((), ())),
        preferred_element_type=jnp.float32)
    d = (x2 + w2[None, :]) - 2.0 * mm
    dmin = jnp.min(d, axis=1, keepdims=True)
    kiota = jax.lax.broadcasted_iota(jnp.int32, d.shape, 1)
    idx = jnp.min(jnp.where(d == dmin, kiota, _K), axis=1)
    enc = (kiota == idx[:, None]).astype(jnp.float32)
    enc_ref[...] = enc
    idx_ref[...] = idx.reshape(1, 1, _ROWS)
    q = jax.lax.dot_general(
        enc.astype(jnp.bfloat16), w.astype(jnp.bfloat16),
        dimension_numbers=(((1,), (0,)), ((), ())),
        preferred_element_type=jnp.float32)
    q_ref[...] = q
    part = jnp.sum((q - f) ** 2, keepdims=True).reshape(1, 1)
    ls_ref[...] = jnp.broadcast_to(part, (1, 1, 128))


@functools.partial(jax.jit, static_argnames=())
def kernel(x, weight):
    n = x.shape[0] * x.shape[2] * x.shape[3]
    grid = n // _ROWS
    xp = jnp.transpose(x, (0, 2, 3, 1))
    flat = xp.reshape(n, _D)
    enc, idx3, q, ls = pl.pallas_call(
        _vq_tile,
        grid=(grid,),
        in_specs=[
            pl.BlockSpec((_ROWS, _D), lambda i: (i, 0)),
            pl.BlockSpec((_K, _D), lambda i: (0, 0)),
        ],
        out_specs=[
            pl.BlockSpec((_ROWS, _K), lambda i: (i, 0)),
            pl.BlockSpec((1, 1, _ROWS), lambda i: (i, 0, 0)),
            pl.BlockSpec((_ROWS, _D), lambda i: (i, 0)),
            pl.BlockSpec((1, 1, 128), lambda i: (i, 0, 0)),
        ],
        out_shape=[
            jax.ShapeDtypeStruct((n, _K), jnp.float32),
            jax.ShapeDtypeStruct((grid, 1, _ROWS), jnp.int32),
            jax.ShapeDtypeStruct((n, _D), jnp.float32),
            jax.ShapeDtypeStruct((grid, 1, 128), jnp.float32),
        ],
        compiler_params=pltpu.CompilerParams(
            dimension_semantics=("parallel",),
        ),
    )(flat, weight)
    loss = jnp.sum(ls[:, 0, 0]) * (2.0 / (n * _D))
    quantized_st = jnp.transpose(q.reshape(xp.shape), (0, 3, 1, 2))
    return loss, quantized_st, enc, idx3.reshape(n)
